# baseline (speedup 1.0000x reference)
import functools

import jax
import jax.numpy as jnp
from jax import lax
from jax.experimental import pallas as pl
from jax.experimental.pallas import tpu as pltpu
from jax.experimental.pallas import tpu_sc as plsc

_C = 256
_NBUF = 4


def _make_sc_gather(B, V, D, nc, ns):
    NW = nc * ns
    b_per_w = B // NW
    C = _C
    NBUF = _NBUF
    nchunks = b_per_w // C
    assert nchunks % NBUF == 0
    mesh = plsc.VectorSubcoreMesh(core_axis_name="c", subcore_axis_name="s")

    @functools.partial(
        pl.kernel,
        out_type=jax.ShapeDtypeStruct((B, D), jnp.float32),
        mesh=mesh,
        scratch_types=(
            [pltpu.VMEM((b_per_w,), jnp.int32)]
            + [pltpu.VMEM((C, D), jnp.float32) for _ in range(NBUF)]
            + [pltpu.SemaphoreType.DMA for _ in range(2 * NBUF)]
        ),
        compiler_params=pltpu.CompilerParams(use_tc_tiling_on_sc=False),
    )
    def emb(idx_hbm, w_hbm, out_hbm, idx_v, *bufs_and_sems):
        bufs = bufs_and_sems[:NBUF]
        gsem = bufs_and_sems[NBUF:2 * NBUF]
        wsem = bufs_and_sems[2 * NBUF:]
        wid = lax.axis_index("s") * nc + lax.axis_index("c")
        base = wid * b_per_w
        pltpu.sync_copy(idx_hbm.at[pl.ds(base, b_per_w)], idx_v)

        def gather(b, c):
            return pltpu.make_async_copy(
                w_hbm.at[idx_v.at[pl.ds(c * C, C)]], bufs[b], gsem[b])

        def write(b, c):
            return pltpu.make_async_copy(
                bufs[b], out_hbm.at[pl.ds(base + c * C, C)], wsem[b])

        for b in range(NBUF):
            gather(b, b).start()

        def body(o, _):
            for b in range(NBUF):
                c = o * NBUF + b
                gather(b, c).wait()
                write(b, c).start()

                @pl.when(c + NBUF < nchunks)
                def _():
                    write(b, c).wait()
                    gather(b, c + NBUF).start()

            return 0

        lax.fori_loop(0, nchunks // NBUF, body, 0)
        for b in range(NBUF):
            write(b, nchunks - NBUF + b).wait()

    return emb


def kernel(input_tensor, weight):
    R, S = input_tensor.shape
    V, D = weight.shape
    B = R * S
    idx_flat = input_tensor.reshape(B).astype(jnp.int32)
    info = plsc.get_sparse_core_info()
    emb = _make_sc_gather(B, V, D, info.num_cores, info.num_subcores)
    out = emb(idx_flat, weight)
    return out.reshape(R, S, D)
